# Initial kernel scaffold; baseline (speedup 1.0000x reference)
import jax
import jax.numpy as jnp
from jax.experimental import pallas as pl


def kernel(idx, token_table, pos_table):
    raise NotImplementedError("write your pallas kernel here")


# speedup vs baseline: 3.8125x; 3.8125x over previous
import functools

import jax
import jax.numpy as jnp
from jax import lax
from jax.experimental import pallas as pl
from jax.experimental.pallas import tpu as pltpu
from jax.experimental.pallas import tpu_sc as plsc

D = 128
B = 1024
T = 200
ROWS = B * T
NC = 2
NS = 16
L = 16
NW = NC * NS
RPW = ROWS // NW
CH = 200
G = 100
NG = CH // G
NCH = RPW // CH


def _body(idx_hbm, tok_hbm, pos_hbm, out_hbm, idx_v, pos_v, buf, semg):
  wid = lax.axis_index("s") * NC + lax.axis_index("c")
  pltpu.sync_copy(idx_hbm.at[pl.ds(wid * NCH, NCH)], idx_v)
  pltpu.sync_copy(pos_hbm, pos_v)

  def chunk_body(j, carry):
    for g in range(NG):
      pltpu.async_copy(
          tok_hbm.at[idx_v.at[j, g]], buf.at[pl.ds(g * G, G)], semg
      ).wait()

    def add_row(r, c2):
      for c in range(D // L):
        s = pl.ds(c * L, L)
        buf[r, s] = buf[r, s] + pos_v[r, s]
      return c2

    lax.fori_loop(0, CH, add_row, 0)
    pltpu.sync_copy(buf, out_hbm.at[pl.ds(wid * RPW + j * CH, CH)])
    return carry

  lax.fori_loop(0, NCH, chunk_body, 0)


_mesh = plsc.VectorSubcoreMesh(core_axis_name="c", subcore_axis_name="s")

_call = functools.partial(
    pl.kernel,
    mesh=_mesh,
    out_type=jax.ShapeDtypeStruct((ROWS, D), jnp.float32),
    scratch_types=[
        pltpu.VMEM((NCH, NG, G), jnp.int32),
        pltpu.VMEM((T, D), jnp.float32),
        pltpu.VMEM((CH, D), jnp.float32),
        pltpu.SemaphoreType.DMA,
    ],
)(_body)


@jax.jit
def kernel(idx, token_table, pos_table):
  idx2 = idx.reshape(NW * NCH, NG, G).astype(jnp.int32)
  out = _call(idx2, token_table, pos_table[:T])
  return out.reshape(B, T, D)
